# Initial kernel scaffold; baseline (speedup 1.0000x reference)
import jax
import jax.numpy as jnp
from jax.experimental import pallas as pl


def kernel(x, edge_index, W1, b1, p1, W2, b2, p2, W3, b3, p3, L1w, L1b, L2w, L2b, L3w, L3b):
    raise NotImplementedError("write your pallas kernel here")


# speedup vs baseline: 1.0080x; 1.0080x over previous
import functools

import jax
import jax.numpy as jnp
from jax import lax
from jax.experimental import pallas as pl
from jax.experimental.pallas import tpu as pltpu
from jax.experimental.pallas import tpu_sc as plsc

N1 = 50000
E0 = 800000
K1, K2, K3 = 40000, 32000, 25600
NP1, NP2, NP3 = 51200, 40960, 32768
EB = 128
E_PAD = 819200
NC, NS = 2, 16


@functools.cache
def _edge_sum_kernel(n_pad: int, feat: int):
    epw = E_PAD // (NC * NS)
    nb = epw // EB
    rows_per_tile = n_pad // NS
    nrb = rows_per_tile // EB
    mesh = plsc.VectorSubcoreMesh(core_axis_name="c", subcore_axis_name="s")

    @functools.partial(
        pl.kernel,
        mesh=mesh,
        compiler_params=pltpu.CompilerParams(use_tc_tiling_on_sc=False),
        out_type=jax.ShapeDtypeStruct((NC * n_pad, feat), jnp.float32),
        scratch_types=[
            pltpu.VMEM_SHARED((n_pad, feat), jnp.float32),
            pltpu.VMEM((8 * EB,), jnp.int32),
            pltpu.VMEM((8, EB), jnp.int32),
            pltpu.VMEM((EB, feat), jnp.float32),
            pltpu.VMEM((EB, feat), jnp.float32),
            pltpu.SemaphoreType.DMA,
        ],
    )
    def k(table, src, dst, zblk, out, acc, srcv, dstv, rows, buf, sem):
        c = lax.axis_index("c")
        s = lax.axis_index("s")
        tid = c * NS + s
        ebase = tid * epw
        rbase = s * rows_per_tile

        pltpu.sync_copy(zblk, buf)

        def zero_body(i, carry):
            pltpu.sync_copy(buf, acc.at[pl.ds(rbase + i * EB, EB)])
            return carry

        lax.fori_loop(0, nrb, zero_body, 0)
        plsc.subcore_barrier()

        def group_body(g, carry):
            pltpu.sync_copy(src.at[pl.ds(ebase + g * 8 * EB, 8 * EB)], srcv)
            pltpu.sync_copy(dst.at[pl.ds(tid * nb + g * 8, 8)], dstv)

            def edge_body(j, carry2):
                pltpu.async_copy(
                    table.at[srcv.at[pl.ds(j * EB, EB)]], rows, sem
                ).wait()
                pltpu.sync_copy(rows, acc.at[dstv.at[j]], add=True)
                return carry2

            lax.fori_loop(0, 8, edge_body, 0)
            return carry

        lax.fori_loop(0, nb // 8, group_body, 0)
        plsc.subcore_barrier()

        def out_body(i, carry):
            pltpu.sync_copy(acc.at[pl.ds(rbase + i * EB, EB)], buf)
            pltpu.sync_copy(
                buf, out.at[pl.ds(c * n_pad + rbase + i * EB, EB)]
            )
            return carry

        lax.fori_loop(0, nrb, out_body, 0)

    return k


def _edge_sum(table, src, dst2d, n_pad, feat):
    zblk = jnp.zeros((EB, feat), jnp.float32)
    out = _edge_sum_kernel(n_pad, feat)(table, src, dst2d, zblk)
    return out[:n_pad] + out[n_pad:]


@functools.cache
def _gcn_tc(n: int, feat: int, blk: int):
    def body(agg_ref, xin_ref, dis_ref, w_ref, b_ref, p_ref, h_ref, s_ref):
        dis = dis_ref[...]
        pre = agg_ref[...] * dis + xin_ref[...] * (dis * dis)
        h = jnp.maximum(
            jnp.dot(pre, w_ref[...], preferred_element_type=jnp.float32)
            + b_ref[...],
            0.0,
        )
        h_ref[...] = h
        s_ref[...] = jnp.tanh(
            jnp.dot(h, p_ref[...], preferred_element_type=jnp.float32)
        )

    return pl.pallas_call(
        body,
        grid=(n // blk,),
        in_specs=[
            pl.BlockSpec((blk, feat), lambda i: (i, 0)),
            pl.BlockSpec((blk, feat), lambda i: (i, 0)),
            pl.BlockSpec((blk, 1), lambda i: (i, 0)),
            pl.BlockSpec((feat, 128), lambda i: (0, 0)),
            pl.BlockSpec((1, 128), lambda i: (0, 0)),
            pl.BlockSpec((128, 1), lambda i: (0, 0)),
        ],
        out_specs=[
            pl.BlockSpec((blk, 128), lambda i: (i, 0)),
            pl.BlockSpec((blk, 1), lambda i: (i, 0)),
        ],
        out_shape=[
            jax.ShapeDtypeStruct((n, 128), jnp.float32),
            jax.ShapeDtypeStruct((n, 1), jnp.float32),
        ],
    )


@functools.cache
def _readout_tc(k: int, blk: int):
    def body(x_ref, mx_ref, sm_ref):
        i = pl.program_id(0)
        bm = jnp.max(x_ref[...], axis=0, keepdims=True)
        bs = jnp.sum(x_ref[...], axis=0, keepdims=True)

        @pl.when(i == 0)
        def _():
            mx_ref[...] = bm
            sm_ref[...] = bs

        @pl.when(i != 0)
        def _():
            mx_ref[...] = jnp.maximum(mx_ref[...], bm)
            sm_ref[...] = sm_ref[...] + bs

    return pl.pallas_call(
        body,
        grid=(k // blk,),
        in_specs=[pl.BlockSpec((blk, 128), lambda i: (i, 0))],
        out_specs=[
            pl.BlockSpec((1, 128), lambda i: (0, 0)),
            pl.BlockSpec((1, 128), lambda i: (0, 0)),
        ],
        out_shape=[
            jax.ShapeDtypeStruct((1, 128), jnp.float32),
            jax.ShapeDtypeStruct((1, 128), jnp.float32),
        ],
    )


def _readout(xn, k, blk):
    mx, sm = _readout_tc(k, blk)(xn)
    return jnp.concatenate([mx, sm / float(k)], axis=1)


def _mlp_head(z, l1w, l1b, l2w, l2b, l3w, l3b):
    def body(z_ref, w1, b1, w2, b2, w3, b3, o_ref):
        t = jnp.maximum(
            jnp.dot(z_ref[...], w1[...], preferred_element_type=jnp.float32)
            + b1[...],
            0.0,
        )
        t = jnp.maximum(
            jnp.dot(t, w2[...], preferred_element_type=jnp.float32) + b2[...],
            0.0,
        )
        o_ref[...] = jax.nn.sigmoid(
            jnp.dot(t, w3[...], preferred_element_type=jnp.float32) + b3[...]
        )

    return pl.pallas_call(
        body, out_shape=jax.ShapeDtypeStruct((1, 124), jnp.float32)
    )(
        z,
        l1w,
        l1b.reshape(1, -1),
        l2w,
        l2b.reshape(1, -1),
        l3w,
        l3b.reshape(1, -1),
    )


def _pool(h, score, src, dst, n, k):
    vals, perm = lax.top_k(score, k)
    xn = h[perm] * vals[:, None]
    newid = (
        jnp.full((n,), -1, jnp.int32)
        .at[perm]
        .set(jnp.arange(k, dtype=jnp.int32))
    )
    newid_ext = jnp.concatenate([newid, jnp.full((1,), -1, jnp.int32)])
    vs = newid_ext[src]
    vd = newid_ext[dst]
    valid = (vs >= 0) & (vd >= 0)
    srcn = jnp.where(valid, vs, k)
    dstn = jnp.where(valid, vd, k)
    return xn, srcn, dstn


def _gcn_layer_sc(xin, src, dst2d, w, b, phat, n, n_pad):
    ones_t = jnp.zeros((n_pad, 8), jnp.float32).at[:n].set(1.0)
    deg = _edge_sum(ones_t, src, dst2d, n_pad, 8)[:n, 0]
    dis = lax.rsqrt(deg + 1.0)
    xs = xin * dis[:, None]
    tbl = (
        jnp.zeros((4, n_pad, 32), jnp.float32)
        .at[:, :n]
        .set(xs.reshape(n, 4, 32).transpose(1, 0, 2))
    )
    chunks = [_edge_sum(tbl[j], src, dst2d, n_pad, 32)[:n] for j in range(4)]
    agg = jnp.stack(chunks, axis=1).reshape(n, 128)
    h, s = _gcn_tc(n, 128, 2000)(
        agg,
        xin,
        dis.reshape(-1, 1),
        w,
        b.reshape(1, -1),
        phat.reshape(-1, 1),
    )
    return h, s[:, 0]


def kernel(x, edge_index, W1, b1, p1, W2, b2, p2, W3, b3, p3,
           L1w, L1b, L2w, L2b, L3w, L3b):
    pad = jnp.full((E_PAD - E0,), N1, jnp.int32)
    src0 = jnp.concatenate([edge_index[0], pad])
    dst0 = jnp.concatenate([edge_index[1], pad])

    dst0_2d = dst0.reshape(-1, EB)
    ones1 = jnp.zeros((NP1, 8), jnp.float32).at[:N1].set(1.0)
    deg1 = _edge_sum(ones1, src0, dst0_2d, NP1, 8)[:N1, 0]
    dis1 = lax.rsqrt(deg1 + 1.0)
    x8 = jnp.zeros((N1, 8), jnp.float32).at[:, :4].set(x)
    t1 = jnp.zeros((NP1, 8), jnp.float32).at[:N1].set(x8 * dis1[:, None])
    agg1 = _edge_sum(t1, src0, dst0_2d, NP1, 8)[:N1]
    w1p = jnp.zeros((8, 128), jnp.float32).at[:4].set(W1)
    h1, s1 = _gcn_tc(N1, 8, 2000)(
        agg1,
        x8,
        dis1.reshape(-1, 1),
        w1p,
        b1.reshape(1, -1),
        (p1 / jnp.linalg.norm(p1)).reshape(-1, 1),
    )
    xn1, src1, dst1 = _pool(h1, s1[:, 0], src0, dst0, N1, K1)
    r1 = _readout(xn1, K1, 1600)

    h2, s2 = _gcn_layer_sc(
        xn1, src1, dst1.reshape(-1, EB), W2, b2,
        p2 / jnp.linalg.norm(p2), K1, NP2,
    )
    xn2, src2, dst2 = _pool(h2, s2, src1, dst1, K1, K2)
    r2 = _readout(xn2, K2, 1600)

    h3, s3 = _gcn_layer_sc(
        xn2, src2, dst2.reshape(-1, EB), W3, b3,
        p3 / jnp.linalg.norm(p3), K2, NP3,
    )
    xn3, _, _ = _pool(h3, s3, src2, dst2, K2, K3)
    r3 = _readout(xn3, K3, 1600)

    z = r1 + r2 + r3
    return _mlp_head(z, L1w, L1b, L2w, L2b, L3w, L3b)


# speedup vs baseline: 1.0631x; 1.0546x over previous
import functools

import jax
import jax.numpy as jnp
from jax import lax
from jax.experimental import pallas as pl
from jax.experimental.pallas import tpu as pltpu
from jax.experimental.pallas import tpu_sc as plsc

N1 = 50000
E0 = 800000
K1, K2, K3 = 40000, 32000, 25600
NP1, NP2, NP3 = 51200, 40960, 32768
EB = 128
E_PAD = 819200
NC, NS = 2, 16


@functools.cache
def _edge_sum_kernel(n_pad: int, feat: int):
    epw = E_PAD // (NC * NS)
    nb = epw // EB
    rows_per_tile = n_pad // NS
    nrb = rows_per_tile // EB
    mesh = plsc.VectorSubcoreMesh(core_axis_name="c", subcore_axis_name="s")

    @functools.partial(
        pl.kernel,
        mesh=mesh,
        compiler_params=pltpu.CompilerParams(use_tc_tiling_on_sc=False),
        out_type=jax.ShapeDtypeStruct((NC * n_pad, feat), jnp.float32),
        scratch_types=[
            pltpu.VMEM_SHARED((n_pad, feat), jnp.float32),
            pltpu.VMEM((8 * EB,), jnp.int32),
            pltpu.VMEM((8, EB), jnp.int32),
            [pltpu.VMEM((EB, feat), jnp.float32) for _ in range(8)],
            pltpu.VMEM((EB, feat), jnp.float32),
            pltpu.SemaphoreType.DMA,
            pltpu.SemaphoreType.DMA,
        ],
    )
    def k(table, src, dst, zblk, out, acc, srcv, dstv, rows, buf, gsem, ssem):
        c = lax.axis_index("c")
        s = lax.axis_index("s")
        tid = c * NS + s
        ebase = tid * epw
        rbase = s * rows_per_tile

        pltpu.sync_copy(zblk, buf)

        def zero_body(i, carry):
            pltpu.sync_copy(buf, acc.at[pl.ds(rbase + i * EB, EB)])
            return carry

        lax.fori_loop(0, nrb, zero_body, 0)
        plsc.subcore_barrier()

        def group_body(g, carry):
            pltpu.sync_copy(src.at[pl.ds(ebase + g * 8 * EB, 8 * EB)], srcv)
            pltpu.sync_copy(dst.at[pl.ds(tid * nb + g * 8, 8)], dstv)
            gds = [
                pltpu.async_copy(
                    table.at[srcv.at[pl.ds(j * EB, EB)]], rows[j], gsem
                )
                for j in range(8)
            ]
            sds = []
            for j in range(8):
                gds[j].wait()
                sds.append(
                    pltpu.async_copy(
                        rows[j], acc.at[dstv.at[j]], ssem, add=True
                    )
                )
            for d in sds:
                d.wait()
            return carry

        lax.fori_loop(0, nb // 8, group_body, 0)
        plsc.subcore_barrier()

        def out_body(i, carry):
            pltpu.sync_copy(acc.at[pl.ds(rbase + i * EB, EB)], buf)
            pltpu.sync_copy(
                buf, out.at[pl.ds(c * n_pad + rbase + i * EB, EB)]
            )
            return carry

        lax.fori_loop(0, nrb, out_body, 0)

    return k


def _edge_sum(table, src, dst2d, n_pad, feat):
    zblk = jnp.zeros((EB, feat), jnp.float32)
    out = _edge_sum_kernel(n_pad, feat)(table, src, dst2d, zblk)
    return out[:n_pad] + out[n_pad:]


@functools.cache
def _edge_count_kernel(n_pad: int):
    epw = E_PAD // (NC * NS)
    nb = epw // EB
    rows_per_tile = n_pad // NS
    nrb = rows_per_tile // EB
    mesh = plsc.VectorSubcoreMesh(core_axis_name="c", subcore_axis_name="s")

    @functools.partial(
        pl.kernel,
        mesh=mesh,
        compiler_params=pltpu.CompilerParams(use_tc_tiling_on_sc=False),
        out_type=jax.ShapeDtypeStruct((NC * n_pad, 8), jnp.float32),
        scratch_types=[
            pltpu.VMEM_SHARED((n_pad, 8), jnp.float32),
            pltpu.VMEM((8, EB), jnp.int32),
            pltpu.VMEM((EB, 8), jnp.float32),
            pltpu.VMEM((EB, 8), jnp.float32),
            pltpu.SemaphoreType.DMA,
        ],
    )
    def k(dst, zblk, oblk, out, acc, dstv, onesv, buf, ssem):
        c = lax.axis_index("c")
        s = lax.axis_index("s")
        tid = c * NS + s
        rbase = s * rows_per_tile

        pltpu.sync_copy(zblk, buf)
        pltpu.sync_copy(oblk, onesv)

        def zero_body(i, carry):
            pltpu.sync_copy(buf, acc.at[pl.ds(rbase + i * EB, EB)])
            return carry

        lax.fori_loop(0, nrb, zero_body, 0)
        plsc.subcore_barrier()

        def group_body(g, carry):
            pltpu.sync_copy(dst.at[pl.ds(tid * nb + g * 8, 8)], dstv)
            sds = [
                pltpu.async_copy(onesv, acc.at[dstv.at[j]], ssem, add=True)
                for j in range(8)
            ]
            for d in sds:
                d.wait()
            return carry

        lax.fori_loop(0, nb // 8, group_body, 0)
        plsc.subcore_barrier()

        def out_body(i, carry):
            pltpu.sync_copy(acc.at[pl.ds(rbase + i * EB, EB)], buf)
            pltpu.sync_copy(
                buf, out.at[pl.ds(c * n_pad + rbase + i * EB, EB)]
            )
            return carry

        lax.fori_loop(0, nrb, out_body, 0)

    return k


def _edge_count(dst2d, n_pad):
    zblk = jnp.zeros((EB, 8), jnp.float32)
    oblk = jnp.ones((EB, 8), jnp.float32)
    out = _edge_count_kernel(n_pad)(dst2d, zblk, oblk)
    return out[:n_pad, 0] + out[n_pad:, 0]


@functools.cache
def _gcn_tc(n: int, feat: int, blk: int):
    def body(agg_ref, xin_ref, dis_ref, w_ref, b_ref, p_ref, h_ref, s_ref):
        dis = dis_ref[...]
        pre = agg_ref[...] * dis + xin_ref[...] * (dis * dis)
        h = jnp.maximum(
            jnp.dot(pre, w_ref[...], preferred_element_type=jnp.float32)
            + b_ref[...],
            0.0,
        )
        h_ref[...] = h
        s_ref[...] = jnp.tanh(
            jnp.dot(h, p_ref[...], preferred_element_type=jnp.float32)
        )

    return pl.pallas_call(
        body,
        grid=(n // blk,),
        in_specs=[
            pl.BlockSpec((blk, feat), lambda i: (i, 0)),
            pl.BlockSpec((blk, feat), lambda i: (i, 0)),
            pl.BlockSpec((blk, 1), lambda i: (i, 0)),
            pl.BlockSpec((feat, 128), lambda i: (0, 0)),
            pl.BlockSpec((1, 128), lambda i: (0, 0)),
            pl.BlockSpec((128, 1), lambda i: (0, 0)),
        ],
        out_specs=[
            pl.BlockSpec((blk, 128), lambda i: (i, 0)),
            pl.BlockSpec((blk, 1), lambda i: (i, 0)),
        ],
        out_shape=[
            jax.ShapeDtypeStruct((n, 128), jnp.float32),
            jax.ShapeDtypeStruct((n, 1), jnp.float32),
        ],
    )


@functools.cache
def _readout_tc(k: int, blk: int):
    def body(x_ref, mx_ref, sm_ref):
        i = pl.program_id(0)
        bm = jnp.max(x_ref[...], axis=0, keepdims=True)
        bs = jnp.sum(x_ref[...], axis=0, keepdims=True)

        @pl.when(i == 0)
        def _():
            mx_ref[...] = bm
            sm_ref[...] = bs

        @pl.when(i != 0)
        def _():
            mx_ref[...] = jnp.maximum(mx_ref[...], bm)
            sm_ref[...] = sm_ref[...] + bs

    return pl.pallas_call(
        body,
        grid=(k // blk,),
        in_specs=[pl.BlockSpec((blk, 128), lambda i: (i, 0))],
        out_specs=[
            pl.BlockSpec((1, 128), lambda i: (0, 0)),
            pl.BlockSpec((1, 128), lambda i: (0, 0)),
        ],
        out_shape=[
            jax.ShapeDtypeStruct((1, 128), jnp.float32),
            jax.ShapeDtypeStruct((1, 128), jnp.float32),
        ],
    )


def _readout(xn, k, blk):
    mx, sm = _readout_tc(k, blk)(xn)
    return jnp.concatenate([mx, sm / float(k)], axis=1)


def _mlp_head(z, l1w, l1b, l2w, l2b, l3w, l3b):
    def body(z_ref, w1, b1, w2, b2, w3, b3, o_ref):
        t = jnp.maximum(
            jnp.dot(z_ref[...], w1[...], preferred_element_type=jnp.float32)
            + b1[...],
            0.0,
        )
        t = jnp.maximum(
            jnp.dot(t, w2[...], preferred_element_type=jnp.float32) + b2[...],
            0.0,
        )
        o_ref[...] = jax.nn.sigmoid(
            jnp.dot(t, w3[...], preferred_element_type=jnp.float32) + b3[...]
        )

    return pl.pallas_call(
        body, out_shape=jax.ShapeDtypeStruct((1, 124), jnp.float32)
    )(
        z,
        l1w,
        l1b.reshape(1, -1),
        l2w,
        l2b.reshape(1, -1),
        l3w,
        l3b.reshape(1, -1),
    )


def _pool(h, score, src, dst, n, k):
    vals, perm = lax.top_k(score, k)
    xn = h[perm] * vals[:, None]
    newid = (
        jnp.full((n,), -1, jnp.int32)
        .at[perm]
        .set(jnp.arange(k, dtype=jnp.int32))
    )
    newid_ext = jnp.concatenate([newid, jnp.full((1,), -1, jnp.int32)])
    vs = newid_ext[src]
    vd = newid_ext[dst]
    valid = (vs >= 0) & (vd >= 0)
    srcn = jnp.where(valid, vs, k)
    dstn = jnp.where(valid, vd, k)
    return xn, srcn, dstn


def _gcn_layer_sc(xin, src, dst2d, w, b, phat, n, n_pad):
    deg = _edge_count(dst2d, n_pad)[:n]
    dis = lax.rsqrt(deg + 1.0)
    xs = xin * dis[:, None]
    tbl = (
        jnp.zeros((4, n_pad, 32), jnp.float32)
        .at[:, :n]
        .set(xs.reshape(n, 4, 32).transpose(1, 0, 2))
    )
    chunks = [_edge_sum(tbl[j], src, dst2d, n_pad, 32)[:n] for j in range(4)]
    agg = jnp.stack(chunks, axis=1).reshape(n, 128)
    h, s = _gcn_tc(n, 128, 2000)(
        agg,
        xin,
        dis.reshape(-1, 1),
        w,
        b.reshape(1, -1),
        phat.reshape(-1, 1),
    )
    return h, s[:, 0]


def kernel(x, edge_index, W1, b1, p1, W2, b2, p2, W3, b3, p3,
           L1w, L1b, L2w, L2b, L3w, L3b):
    pad = jnp.full((E_PAD - E0,), N1, jnp.int32)
    src0 = jnp.concatenate([edge_index[0], pad])
    dst0 = jnp.concatenate([edge_index[1], pad])

    dst0_2d = dst0.reshape(-1, EB)
    deg1 = _edge_count(dst0_2d, NP1)[:N1]
    dis1 = lax.rsqrt(deg1 + 1.0)
    x8 = jnp.zeros((N1, 8), jnp.float32).at[:, :4].set(x)
    t1 = jnp.zeros((NP1, 8), jnp.float32).at[:N1].set(x8 * dis1[:, None])
    agg1 = _edge_sum(t1, src0, dst0_2d, NP1, 8)[:N1]
    w1p = jnp.zeros((8, 128), jnp.float32).at[:4].set(W1)
    h1, s1 = _gcn_tc(N1, 8, 2000)(
        agg1,
        x8,
        dis1.reshape(-1, 1),
        w1p,
        b1.reshape(1, -1),
        (p1 / jnp.linalg.norm(p1)).reshape(-1, 1),
    )
    xn1, src1, dst1 = _pool(h1, s1[:, 0], src0, dst0, N1, K1)
    r1 = _readout(xn1, K1, 1600)

    h2, s2 = _gcn_layer_sc(
        xn1, src1, dst1.reshape(-1, EB), W2, b2,
        p2 / jnp.linalg.norm(p2), K1, NP2,
    )
    xn2, src2, dst2 = _pool(h2, s2, src1, dst1, K1, K2)
    r2 = _readout(xn2, K2, 1600)

    h3, s3 = _gcn_layer_sc(
        xn2, src2, dst2.reshape(-1, EB), W3, b3,
        p3 / jnp.linalg.norm(p3), K2, NP3,
    )
    xn3, _, _ = _pool(h3, s3, src2, dst2, K2, K3)
    r3 = _readout(xn3, K3, 1600)

    z = r1 + r2 + r3
    return _mlp_head(z, L1w, L1b, L2w, L2b, L3w, L3b)
